# baseline (speedup 1.0000x reference)
import functools

import jax
import jax.numpy as jnp
from jax import lax
from jax.experimental import pallas as pl
from jax.experimental.pallas import tpu as pltpu
from jax.experimental.pallas import tpu_sc as plsc

NUM_FEATURES = 26
VOCAB = 16
LANES = 16
PERIOD = NUM_FEATURES * LANES // 2
VREGS_PER_PERIOD = PERIOD // LANES


@functools.lru_cache(maxsize=None)
def _make_lookup(total: int):
    info = plsc.get_sparse_core_info()
    nw = info.num_cores * info.num_subcores
    assert total % (nw * PERIOD) == 0
    chunk = total // nw
    groups = chunk // PERIOD

    mesh = plsc.VectorSubcoreMesh(core_axis_name="c", subcore_axis_name="s")

    @functools.partial(
        pl.kernel,
        mesh=mesh,
        out_type=jax.ShapeDtypeStruct((total,), jnp.float32),
        scratch_types=[
            pltpu.VMEM((chunk,), jnp.int32),
            pltpu.VMEM((chunk,), jnp.float32),
            pltpu.VMEM((NUM_FEATURES * VOCAB,), jnp.float32),
            pltpu.VMEM((PERIOD,), jnp.int32),
        ],
        compiler_params=pltpu.CompilerParams(
            needs_layout_passes=False, disable_bounds_checks=True
        ),
    )
    def lookup(tok_hbm, tbl_hbm, out_hbm, tok_v, out_v, tbl_v, offs_v):
        wid = lax.axis_index("s") * info.num_cores + lax.axis_index("c")
        base = wid * chunk
        pltpu.sync_copy(tok_hbm.at[pl.ds(base, chunk)], tok_v)
        pltpu.sync_copy(tbl_hbm, tbl_v)
        for j in range(VREGS_PER_PERIOD):
            p = lax.iota(jnp.int32, LANES) + (j * LANES)
            offs_v[pl.ds(j * LANES, LANES)] = lax.rem(p, NUM_FEATURES) * VOCAB

        def body(g, carry):
            go = g * PERIOD
            for j in range(VREGS_PER_PERIOD):
                o = go + j * LANES
                idx = tok_v[pl.ds(o, LANES)] + offs_v[pl.ds(j * LANES, LANES)]
                out_v[pl.ds(o, LANES)] = plsc.load_gather(tbl_v, [idx])
            return carry

        lax.fori_loop(0, groups, body, 0)
        pltpu.sync_copy(out_v, out_hbm.at[pl.ds(base, chunk)])

    return lookup


def kernel(inputs, mapping):
    shape = inputs.shape
    tok = inputs.astype(jnp.int32).reshape(-1)
    tbl = mapping.astype(jnp.float32).reshape(-1)
    out = _make_lookup(tok.size)(tok, tbl)
    return out.reshape(shape)


# speedup vs baseline: 1.1261x; 1.1261x over previous
import functools

import jax
import jax.numpy as jnp
from jax import lax
from jax.experimental import pallas as pl
from jax.experimental.pallas import tpu as pltpu
from jax.experimental.pallas import tpu_sc as plsc

NUM_FEATURES = 26
VOCAB = 16
LANES = 16
PERIOD = NUM_FEATURES * LANES // 2
VREGS_PER_PERIOD = PERIOD // LANES


@functools.lru_cache(maxsize=None)
def _make_lookup(total: int):
    info = plsc.get_sparse_core_info()
    nw = info.num_cores * info.num_subcores
    assert total % (nw * PERIOD) == 0
    chunk = total // nw
    groups = chunk // PERIOD

    mesh = plsc.VectorSubcoreMesh(core_axis_name="c", subcore_axis_name="s")

    @functools.partial(
        pl.kernel,
        mesh=mesh,
        out_type=jax.ShapeDtypeStruct((total,), jnp.float32),
        scratch_types=[
            pltpu.VMEM((chunk,), jnp.int32),
            pltpu.VMEM((chunk,), jnp.float32),
            pltpu.VMEM((NUM_FEATURES * VOCAB,), jnp.float32),
        ],
        compiler_params=pltpu.CompilerParams(
            needs_layout_passes=False,
            disable_bounds_checks=True,
            skip_device_barrier=True,
        ),
    )
    def lookup(tok_hbm, tbl_hbm, out_hbm, tok_v, out_v, tbl_v):
        wid = lax.axis_index("s") * info.num_cores + lax.axis_index("c")
        base = wid * chunk
        pltpu.sync_copy(tok_hbm.at[pl.ds(base, chunk)], tok_v)
        pltpu.sync_copy(tbl_hbm, tbl_v)
        offs = [
            lax.rem(lax.iota(jnp.int32, LANES) + j * LANES, NUM_FEATURES)
            * VOCAB
            for j in range(VREGS_PER_PERIOD)
        ]

        @plsc.parallel_loop(0, groups)
        def body(g):
            go = g * PERIOD
            for j in range(VREGS_PER_PERIOD):
                o = go + j * LANES
                idx = tok_v[pl.ds(o, LANES)] + offs[j]
                out_v[pl.ds(o, LANES)] = plsc.load_gather(tbl_v, [idx])

        pltpu.sync_copy(out_v, out_hbm.at[pl.ds(base, chunk)])

    return lookup


def kernel(inputs, mapping):
    shape = inputs.shape
    tok = inputs.astype(jnp.int32).reshape(-1)
    tbl = mapping.astype(jnp.float32).reshape(-1)
    out = _make_lookup(tok.size)(tok, tbl)
    return out.reshape(shape)


# speedup vs baseline: 1.5285x; 1.3574x over previous
import functools

import jax
import jax.numpy as jnp
from jax import lax
from jax.experimental import pallas as pl
from jax.experimental.pallas import tpu as pltpu
from jax.experimental.pallas import tpu_sc as plsc

LANES = 16
PERIOD_UNITS = 13


@functools.lru_cache(maxsize=None)
def _make_lookup(batch: int, nfeat: int, vocab: int):
    info = plsc.get_sparse_core_info()
    nw = info.num_cores * info.num_subcores
    period = PERIOD_UNITS * LANES
    rows_per_period = period // nfeat
    rows = batch // nw
    npiece = 2
    rows_piece = rows // npiece
    assert batch % nw == 0 and (rows_piece * nfeat) % period == 0
    groups = rows_piece * nfeat // period

    mesh = plsc.VectorSubcoreMesh(core_axis_name="c", subcore_axis_name="s")

    @functools.partial(
        pl.kernel,
        mesh=mesh,
        out_type=jax.ShapeDtypeStruct((batch, nfeat), jnp.float32),
        scratch_types=[
            pltpu.VMEM((rows_piece, nfeat), jnp.int32),
            pltpu.VMEM((rows_piece, nfeat), jnp.float32),
            pltpu.VMEM((nfeat, vocab), jnp.float32),
        ],
        compiler_params=pltpu.CompilerParams(
            needs_layout_passes=False,
            disable_bounds_checks=True,
        ),
    )
    def lookup(tok_hbm, tbl_hbm, out_hbm, tok_v, out_v, tbl_v):
        wid = lax.axis_index("s") * info.num_cores + lax.axis_index("c")
        base = wid * rows
        pltpu.sync_copy(tbl_hbm, tbl_v)
        lane_p = [
            lax.iota(jnp.int32, LANES) + j * LANES for j in range(PERIOD_UNITS)
        ]
        rowpat = [lax.div(p, nfeat) for p in lane_p]
        colpat = [lax.rem(p, nfeat) for p in lane_p]

        for piece in range(npiece):
            pbase = base + piece * rows_piece
            pltpu.sync_copy(tok_hbm.at[pl.ds(pbase, rows_piece)], tok_v)

            @plsc.parallel_loop(0, groups)
            def body(g):
                grow = g * rows_per_period
                for j in range(PERIOD_UNITS):
                    row = grow + rowpat[j]
                    tok = plsc.load_gather(tok_v, [row, colpat[j]])
                    vals = plsc.load_gather(tbl_v, [colpat[j], tok])
                    plsc.store_scatter(out_v, [row, colpat[j]], vals)

            pltpu.sync_copy(out_v, out_hbm.at[pl.ds(pbase, rows_piece)])

    return lookup


def kernel(inputs, mapping):
    tok = inputs.astype(jnp.int32)
    tbl = mapping.astype(jnp.float32)
    return _make_lookup(tok.shape[0], tok.shape[1], tbl.shape[1])(tok, tbl)


# speedup vs baseline: 2.2160x; 1.4498x over previous
import functools

import jax
import jax.numpy as jnp
from jax import lax
from jax.experimental import pallas as pl
from jax.experimental.pallas import tpu as pltpu
from jax.experimental.pallas import tpu_sc as plsc

LANES = 16


@functools.lru_cache(maxsize=None)
def _make_lookup(batch: int, nfeat: int, vocab: int):
    info = plsc.get_sparse_core_info()
    nw = info.num_cores * info.num_subcores
    cols = batch // nw
    units = cols // LANES
    assert batch % nw == 0 and cols % LANES == 0

    mesh = plsc.VectorSubcoreMesh(core_axis_name="c", subcore_axis_name="s")

    @functools.partial(
        pl.kernel,
        mesh=mesh,
        out_type=jax.ShapeDtypeStruct((nfeat, batch), jnp.float32),
        scratch_types=[
            pltpu.VMEM((nfeat, cols), jnp.int32),
            pltpu.VMEM((nfeat, cols), jnp.float32),
            pltpu.VMEM((vocab, nfeat), jnp.float32),
        ],
        compiler_params=pltpu.CompilerParams(
            needs_layout_passes=False,
            disable_bounds_checks=True,
        ),
    )
    def lookup(tok_hbm, tbl_hbm, out_hbm, tok_v, out_v, tbl_v):
        wid = lax.axis_index("s") * info.num_cores + lax.axis_index("c")
        base = wid * cols
        pltpu.sync_copy(tok_hbm.at[:, pl.ds(base, cols)], tok_v)
        pltpu.sync_copy(tbl_hbm, tbl_v)

        @plsc.parallel_loop(0, nfeat)
        def body(f):
            fvec = jnp.broadcast_to(f, (LANES,)).astype(jnp.int32)
            for c in range(units):
                o = c * LANES
                tok = tok_v[f, pl.ds(o, LANES)]
                vals = plsc.load_gather(tbl_v, [tok, fvec])
                out_v[f, pl.ds(o, LANES)] = vals

        pltpu.sync_copy(out_v, out_hbm.at[:, pl.ds(base, cols)])

    return lookup


def kernel(inputs, mapping):
    tok = jnp.swapaxes(inputs.astype(jnp.int32), 0, 1)
    tbl = jnp.swapaxes(mapping.astype(jnp.float32), 0, 1)
    out = _make_lookup(inputs.shape[0], inputs.shape[1], mapping.shape[1])(
        tok, tbl
    )
    return jnp.swapaxes(out, 0, 1)
